# baseline (speedup 1.0000x reference)
import jax
import jax.numpy as jnp
from jax.experimental import pallas as pl

_D = 4096
_E = 64
_K = 8
_BLOCK = 512


def _router_body(x_ref, w_ref, idx_ref, probs_ref, scores_ref):
    s = jax.lax.dot_general(
        x_ref[...], w_ref[...], (((1,), (1,)), ((), ())),
        preferred_element_type=jnp.float32,
    )
    scores_ref[...] = s

    fiota = jax.lax.broadcasted_iota(jnp.int32, s.shape, 1).astype(jnp.float32)
    neg_inf = jnp.float32(-jnp.inf)
    big = jnp.float32(_E)
    vals, idxs = [], []
    cur = s
    for k in range(_K):
        m = jnp.max(cur, axis=1, keepdims=True)
        i = jnp.min(jnp.where(cur == m, fiota, big), axis=1, keepdims=True)
        vals.append(m)
        idxs.append(i)
        if k + 1 < _K:
            cur = jnp.where(fiota == i, neg_inf, cur)
    v = jnp.concatenate(vals, axis=1)
    fi = jnp.concatenate(idxs, axis=1)

    e = jnp.exp(v - v[:, 0:1])
    probs_ref[...] = e / jnp.sum(e, axis=1, keepdims=True)
    idx_ref[...] = fi.astype(jnp.int32)


def kernel(x, W):
    tokens = x.shape[0]
    grid = (tokens // _BLOCK,)
    out = pl.pallas_call(
        _router_body,
        grid=grid,
        in_specs=[
            pl.BlockSpec((_BLOCK, _D), lambda i: (i, 0)),
            pl.BlockSpec((_E, _D), lambda i: (0, 0)),
        ],
        out_specs=[
            pl.BlockSpec((_BLOCK, _K), lambda i: (i, 0)),
            pl.BlockSpec((_BLOCK, _K), lambda i: (i, 0)),
            pl.BlockSpec((_BLOCK, _E), lambda i: (i, 0)),
        ],
        out_shape=[
            jax.ShapeDtypeStruct((tokens, _K), jnp.int32),
            jax.ShapeDtypeStruct((tokens, _K), jnp.float32),
            jax.ShapeDtypeStruct((tokens, _E), jnp.float32),
        ],
    )(x, W)
    return (out[0], out[1], out[2])


# speedup vs baseline: 1.1012x; 1.1012x over previous
import jax
import jax.numpy as jnp
from jax.experimental import pallas as pl
from jax.experimental.pallas import tpu as pltpu

_D = 4096
_E = 64
_K = 8
_BLOCK = 1024


def _router_body(x_ref, w_ref, idx_ref, probs_ref, scores_ref):
    s = jax.lax.dot_general(
        x_ref[...], w_ref[...], (((1,), (1,)), ((), ())),
        preferred_element_type=jnp.float32,
    )
    scores_ref[...] = s

    fiota = jax.lax.broadcasted_iota(jnp.int32, s.shape, 1).astype(jnp.float32)
    neg_inf = jnp.float32(-jnp.inf)
    big = jnp.float32(_E)
    vals, idxs = [], []
    cur = s
    for k in range(_K):
        m = jnp.max(cur, axis=1, keepdims=True)
        i = jnp.min(jnp.where(cur == m, fiota, big), axis=1, keepdims=True)
        vals.append(m)
        idxs.append(i)
        if k + 1 < _K:
            cur = jnp.where(fiota == i, neg_inf, cur)
    v = jnp.concatenate(vals, axis=1)
    fi = jnp.concatenate(idxs, axis=1)

    e = jnp.exp(v - v[:, 0:1])
    probs_ref[...] = e / jnp.sum(e, axis=1, keepdims=True)
    idx_ref[...] = fi.astype(jnp.int32)


def kernel(x, W):
    tokens = x.shape[0]
    grid = (tokens // _BLOCK,)
    out = pl.pallas_call(
        _router_body,
        grid=grid,
        in_specs=[
            pl.BlockSpec((_BLOCK, _D), lambda i: (i, 0)),
            pl.BlockSpec((_E, _D), lambda i: (0, 0)),
        ],
        out_specs=[
            pl.BlockSpec((_BLOCK, _K), lambda i: (i, 0)),
            pl.BlockSpec((_BLOCK, _K), lambda i: (i, 0)),
            pl.BlockSpec((_BLOCK, _E), lambda i: (i, 0)),
        ],
        out_shape=[
            jax.ShapeDtypeStruct((tokens, _K), jnp.int32),
            jax.ShapeDtypeStruct((tokens, _K), jnp.float32),
            jax.ShapeDtypeStruct((tokens, _E), jnp.float32),
        ],
        compiler_params=pltpu.CompilerParams(
            dimension_semantics=("parallel",),
        ),
    )(x, W)
    return (out[0], out[1], out[2])


# speedup vs baseline: 1.2899x; 1.1714x over previous
import jax
import jax.numpy as jnp
from jax.experimental import pallas as pl
from jax.experimental.pallas import tpu as pltpu

_D = 4096
_E = 64
_K = 8
_BLOCK = 1024


def _router_body(x_ref, w_ref, idx_ref, probs_ref, scores_ref):
    st = jax.lax.dot_general(
        w_ref[...], x_ref[...], (((1,), (1,)), ((), ())),
        preferred_element_type=jnp.float32,
    )
    scores_ref[...] = st.T

    fiota = jax.lax.broadcasted_iota(jnp.int32, st.shape, 0).astype(jnp.float32)
    neg_inf = jnp.float32(-jnp.inf)
    big = jnp.float32(_E)
    vals, idxs = [], []
    cur = st
    for k in range(_K):
        m = jnp.max(cur, axis=0, keepdims=True)
        i = jnp.min(jnp.where(cur == m, fiota, big), axis=0, keepdims=True)
        vals.append(m)
        idxs.append(i)
        if k + 1 < _K:
            cur = jnp.where(fiota == i, neg_inf, cur)
    v = jnp.concatenate(vals, axis=0)
    fi = jnp.concatenate(idxs, axis=0)

    e = jnp.exp(v - v[0:1])
    p = e / jnp.sum(e, axis=0, keepdims=True)
    probs_ref[...] = p.T
    idx_ref[...] = fi.T.astype(jnp.int32)


def kernel(x, W):
    tokens = x.shape[0]
    grid = (tokens // _BLOCK,)
    out = pl.pallas_call(
        _router_body,
        grid=grid,
        in_specs=[
            pl.BlockSpec((_BLOCK, _D), lambda i: (i, 0)),
            pl.BlockSpec((_E, _D), lambda i: (0, 0)),
        ],
        out_specs=[
            pl.BlockSpec((_BLOCK, _K), lambda i: (i, 0)),
            pl.BlockSpec((_BLOCK, _K), lambda i: (i, 0)),
            pl.BlockSpec((_BLOCK, _E), lambda i: (i, 0)),
        ],
        out_shape=[
            jax.ShapeDtypeStruct((tokens, _K), jnp.int32),
            jax.ShapeDtypeStruct((tokens, _K), jnp.float32),
            jax.ShapeDtypeStruct((tokens, _E), jnp.float32),
        ],
        compiler_params=pltpu.CompilerParams(
            dimension_semantics=("parallel",),
            vmem_limit_bytes=100 * 1024 * 1024,
        ),
    )(x, W)
    return (out[0], out[1], out[2])


# speedup vs baseline: 1.2900x; 1.0001x over previous
import jax
import jax.numpy as jnp
from jax.experimental import pallas as pl
from jax.experimental.pallas import tpu as pltpu

_D = 4096
_E = 64
_K = 8
_BLOCK = 1024


def _router_body(x_lo_ref, x_hi_ref, w_ref, idx_ref, probs_ref, scores_ref):
    w = w_ref[...]
    st = jax.lax.dot_general(
        w[:, : _D // 2], x_lo_ref[...], (((1,), (1,)), ((), ())),
        preferred_element_type=jnp.float32,
    ) + jax.lax.dot_general(
        w[:, _D // 2 :], x_hi_ref[...], (((1,), (1,)), ((), ())),
        preferred_element_type=jnp.float32,
    )
    scores_ref[...] = st.T

    fiota = jax.lax.broadcasted_iota(jnp.int32, st.shape, 0).astype(jnp.float32)
    neg_inf = jnp.float32(-jnp.inf)
    big = jnp.float32(_E)
    vals, idxs = [], []
    cur = st
    for k in range(_K):
        m = jnp.max(cur, axis=0, keepdims=True)
        i = jnp.min(jnp.where(cur == m, fiota, big), axis=0, keepdims=True)
        vals.append(m)
        idxs.append(i)
        if k + 1 < _K:
            cur = jnp.where(fiota == i, neg_inf, cur)
    v = jnp.concatenate(vals, axis=0)
    fi = jnp.concatenate(idxs, axis=0)

    e = jnp.exp(v - v[0:1])
    p = e / jnp.sum(e, axis=0, keepdims=True)
    probs_ref[...] = p.T
    idx_ref[...] = fi.T.astype(jnp.int32)


def kernel(x, W):
    tokens = x.shape[0]
    grid = (tokens // _BLOCK,)
    out = pl.pallas_call(
        _router_body,
        grid=grid,
        in_specs=[
            pl.BlockSpec((_BLOCK, _D // 2), lambda i: (i, 0)),
            pl.BlockSpec((_BLOCK, _D // 2), lambda i: (i, 1)),
            pl.BlockSpec((_E, _D), lambda i: (0, 0)),
        ],
        out_specs=[
            pl.BlockSpec((_BLOCK, _K), lambda i: (i, 0)),
            pl.BlockSpec((_BLOCK, _K), lambda i: (i, 0)),
            pl.BlockSpec((_BLOCK, _E), lambda i: (i, 0)),
        ],
        out_shape=[
            jax.ShapeDtypeStruct((tokens, _K), jnp.int32),
            jax.ShapeDtypeStruct((tokens, _K), jnp.float32),
            jax.ShapeDtypeStruct((tokens, _E), jnp.float32),
        ],
        compiler_params=pltpu.CompilerParams(
            dimension_semantics=("parallel",),
            vmem_limit_bytes=100 * 1024 * 1024,
        ),
    )(x, x, W)
    return (out[0], out[1], out[2])
